# baseline (speedup 1.0000x reference)
import jax
import jax.numpy as jnp
from jax.experimental import pallas as pl
from jax.experimental.pallas import tpu as pltpu


def _lap_kernel(x_ref, out_ref):
    x = x_ref[0]
    rb, w = x.shape
    last_row = x[rb - 1 : rb, :]

    xc = jnp.concatenate([x[:, 1:], x[:, w - 1 :]], axis=1)
    xd = jnp.concatenate([x[1:, :], last_row], axis=0)
    xd1 = jnp.concatenate([xd[:, 1:], xd[:, w - 1 :]], axis=1)

    dx = x - xc
    dy = x - xd
    dxd = xd - xd1
    s = dx * dx + dy * dy + (dxd * (dx - dxd)) * (2.0 / 3.0)
    part = jnp.sum(s, axis=0, keepdims=True) * 3.0

    row0 = dx[0:1, :]
    rowl = dx[rb - 1 : rb, :]
    out_ref[0] = part - row0 * row0 + rowl * rowl


def kernel(f):
    B, C, H, W = f.shape
    n = B * C
    x3 = f.reshape(n, H, W)

    out = pl.pallas_call(
        _lap_kernel,
        grid=(n,),
        in_specs=[pl.BlockSpec((1, H, W), lambda i: (i, 0, 0))],
        out_specs=pl.BlockSpec((1, 1, W), lambda i: (i, 0, 0)),
        out_shape=jax.ShapeDtypeStruct((n, 1, W), f.dtype),
        compiler_params=pltpu.CompilerParams(
            dimension_semantics=("parallel",),
        ),
    )(x3)

    return 2.0 * jnp.sum(out)


# speedup vs baseline: 1.0401x; 1.0401x over previous
import jax
import jax.numpy as jnp
from jax.experimental import pallas as pl
from jax.experimental.pallas import tpu as pltpu


def _lap_kernel(x_ref, out_ref):
    x = x_ref[0]
    rb, w = x.shape
    last_row = x[rb - 1 : rb, :]

    xc = jnp.concatenate([x[:, 1:], x[:, w - 1 :]], axis=1)
    xd = jnp.concatenate([x[1:, :], last_row], axis=0)
    xd1 = jnp.concatenate([xd[:, 1:], xd[:, w - 1 :]], axis=1)

    dx = x - xc
    dy = x - xd
    dxd = xd - xd1
    s = (dx * dx + dy * dy) * 3.0 + dxd * (dx - dxd) * 2.0
    part = jnp.sum(s, axis=0, keepdims=True)

    row0 = dx[0:1, :]
    rowl = dx[rb - 1 : rb, :]
    out_ref[0] = part - row0 * row0 + rowl * rowl


def kernel(f):
    B, C, H, W = f.shape
    n = B * C
    x3 = f.reshape(n, H, W)

    out = pl.pallas_call(
        _lap_kernel,
        grid=(n,),
        in_specs=[pl.BlockSpec((1, H, W), lambda i: (i, 0, 0))],
        out_specs=pl.BlockSpec((1, 1, W), lambda i: (i, 0, 0)),
        out_shape=jax.ShapeDtypeStruct((n, 1, W), f.dtype),
        compiler_params=pltpu.CompilerParams(
            dimension_semantics=("parallel",),
        ),
    )(x3)

    return 2.0 * jnp.sum(out)
